# Initial kernel scaffold; baseline (speedup 1.0000x reference)
import jax
import jax.numpy as jnp
from jax.experimental import pallas as pl


def kernel(x, edge_index, W_l1, b_l1, W_r1, W_l2, b_l2, W_r2):
    raise NotImplementedError("write your pallas kernel here")


# speedup vs baseline: 6.8718x; 6.8718x over previous
import functools

import jax
import jax.numpy as jnp
from jax import lax
from jax.experimental import pallas as pl
from jax.experimental.pallas import tpu as pltpu
from jax.experimental.pallas import tpu_sc as plsc

N = 10000
E = 320000
D = 128

NC = 2
NS = 16
NW = NC * NS

B = 128
NCHUNK = E // B
CHUNKS_PER_TILE = -(-NCHUNK // NW)

N_PAD = 10240
ROWS_PER_TILE_SC = N_PAD // NS


def _sc_agg_body(with_deg, *refs):
    if with_deg:
        (x_hbm, src_hbm, dst_hbm, agg_out, deg_out,
         srcv, dstv, rows, zblk, onesv, agg_sh, deg_sh, sem) = refs
    else:
        (x_hbm, src_hbm, dst_hbm, agg_out,
         srcv, dstv, rows, zblk, onesv, agg_sh, deg_sh, sem) = refs

    cid = lax.axis_index("c")
    sid = lax.axis_index("s")
    wid = sid * NC + cid

    zero16 = jnp.zeros((16,), jnp.float32)
    for r in range(16):
        for c8 in range(D // 16):
            zblk[r, pl.ds(c8 * 16, 16)] = zero16
    one16 = jnp.ones((16,), jnp.float32)
    for c8 in range(B // 16):
        onesv[pl.ds(c8 * 16, 16)] = one16

    zbase = sid * ROWS_PER_TILE_SC

    def zloop(i, carry):
        pltpu.sync_copy(zblk, agg_sh.at[pl.ds(zbase + i * 16, 16)])
        return carry

    lax.fori_loop(0, ROWS_PER_TILE_SC // 16, zloop, 0)
    if with_deg:
        def zdloop(i, carry):
            pltpu.sync_copy(zblk.at[0], deg_sh.at[pl.ds(zbase + i * D, D)])
            return carry

        lax.fori_loop(0, ROWS_PER_TILE_SC // D, zdloop, 0)
    plsc.subcore_barrier()

    def chunk_body(i, carry):
        c = wid + i * NW

        @pl.when(c < NCHUNK)
        def _():
            base = c * B
            pltpu.sync_copy(src_hbm.at[pl.ds(base, B)], srcv)
            pltpu.sync_copy(dst_hbm.at[pl.ds(base, B)], dstv)
            pltpu.async_copy(x_hbm.at[srcv], rows, sem).wait()
            pltpu.sync_copy(rows, agg_sh.at[dstv], add=True)
            if with_deg:
                pltpu.sync_copy(onesv, deg_sh.at[dstv], add=True)

        return carry

    lax.fori_loop(0, CHUNKS_PER_TILE, chunk_body, 0)
    plsc.subcore_barrier()

    pltpu.sync_copy(agg_sh.at[pl.ds(zbase, ROWS_PER_TILE_SC)],
                    agg_out.at[cid, pl.ds(zbase, ROWS_PER_TILE_SC)])
    if with_deg:
        pltpu.sync_copy(deg_sh.at[pl.ds(zbase, ROWS_PER_TILE_SC)],
                        deg_out.at[cid, pl.ds(zbase, ROWS_PER_TILE_SC)])


@functools.lru_cache(maxsize=None)
def _make_sc_agg(with_deg):
    mesh = plsc.VectorSubcoreMesh(core_axis_name="c", subcore_axis_name="s")
    out_type = [jax.ShapeDtypeStruct((NC, N_PAD, D), jnp.float32)]
    if with_deg:
        out_type.append(jax.ShapeDtypeStruct((NC, N_PAD), jnp.float32))
    scratch = [
        pltpu.VMEM((B,), jnp.int32),
        pltpu.VMEM((B,), jnp.int32),
        pltpu.VMEM((B, D), jnp.float32),
        pltpu.VMEM((16, D), jnp.float32),
        pltpu.VMEM((B,), jnp.float32),
        pltpu.VMEM_SHARED((N_PAD, D), jnp.float32),
        pltpu.VMEM_SHARED((N_PAD,), jnp.float32),
        pltpu.SemaphoreType.DMA,
    ]
    return pl.kernel(
        functools.partial(_sc_agg_body, with_deg),
        mesh=mesh,
        out_type=out_type if with_deg else out_type[0],
        scratch_types=scratch,
    )


R_BLK = 1024


def _dense_body(relu, agg_ref, deg_ref, x_ref, wl_ref, b_ref, wr_ref, o_ref):
    agg = agg_ref[0] + agg_ref[1]
    deg = deg_ref[0] + deg_ref[1]
    inv = 1.0 / jnp.maximum(deg, 1.0)
    mean = (agg.reshape(R_BLK // 128, 128, D) * inv[:, :, None]).reshape(R_BLK, D)
    out = jnp.dot(mean, wl_ref[...], preferred_element_type=jnp.float32)
    out = out + jnp.dot(x_ref[...], wr_ref[...], preferred_element_type=jnp.float32)
    out = out + b_ref[...]
    if relu:
        out = jnp.maximum(out, 0.0)
    o_ref[...] = out


def _dense(aggp, degp3, x_pad, wlT, b2, wrT, relu):
    grid = (N_PAD // R_BLK,)
    return pl.pallas_call(
        functools.partial(_dense_body, relu),
        grid=grid,
        in_specs=[
            pl.BlockSpec((NC, R_BLK, D), lambda i: (0, i, 0)),
            pl.BlockSpec((NC, R_BLK // 128, 128), lambda i: (0, i, 0)),
            pl.BlockSpec((R_BLK, D), lambda i: (i, 0)),
            pl.BlockSpec((D, D), lambda i: (0, 0)),
            pl.BlockSpec((1, D), lambda i: (0, 0)),
            pl.BlockSpec((D, D), lambda i: (0, 0)),
        ],
        out_specs=pl.BlockSpec((R_BLK, D), lambda i: (i, 0)),
        out_shape=jax.ShapeDtypeStruct((N_PAD, D), jnp.float32),
    )(aggp, degp3, x_pad, wlT, b2, wrT)


def kernel(x, edge_index, W_l1, b_l1, W_r1, W_l2, b_l2, W_r2):
    src = edge_index[0]
    dst = edge_index[1]
    x_pad = jnp.pad(x, ((0, N_PAD - N), (0, 0)))

    aggp1, degp = _make_sc_agg(True)(x_pad, src, dst)
    degp3 = degp.reshape(NC, N_PAD // 128, 128)
    h = _dense(aggp1, degp3, x_pad, W_l1.T, b_l1.reshape(1, D), W_r1.T, True)

    aggp2 = _make_sc_agg(False)(h, src, dst)
    out = _dense(aggp2, degp3, h, W_l2.T, b_l2.reshape(1, D), W_r2.T, False)
    return out[:N]


# speedup vs baseline: 13.0176x; 1.8944x over previous
import functools

import jax
import jax.numpy as jnp
from jax import lax
from jax.experimental import pallas as pl
from jax.experimental.pallas import tpu as pltpu
from jax.experimental.pallas import tpu_sc as plsc

N = 10000
E = 320000
D = 128

NC = 2
NS = 16
NW = NC * NS

B = 128
NCHUNK = E // B
CHUNKS_PER_TILE = -(-NCHUNK // NW)

N_PAD = 10240
ROWS_PER_TILE_SC = N_PAD // NS


def _sc_agg_body(with_deg, *refs):
    if with_deg:
        (x_hbm, edge_hbm, agg_out, deg_out,
         idxA, idxB, rowsA, rowsB, zblk, onesv, agg_sh, deg_sh,
         semA, semB) = refs
    else:
        (x_hbm, edge_hbm, agg_out,
         idxA, idxB, rowsA, rowsB, zblk, onesv, agg_sh, deg_sh,
         semA, semB) = refs

    cid = lax.axis_index("c")
    sid = lax.axis_index("s")
    wid = sid * NC + cid

    zero16 = jnp.zeros((16,), jnp.float32)
    for r in range(16):
        for c8 in range(D // 16):
            zblk[r, pl.ds(c8 * 16, 16)] = zero16
    one16 = jnp.ones((16,), jnp.float32)
    for c8 in range(B // 16):
        onesv[pl.ds(c8 * 16, 16)] = one16

    zbase = sid * ROWS_PER_TILE_SC

    def zloop(i, carry):
        pltpu.sync_copy(zblk, agg_sh.at[pl.ds(zbase + i * 16, 16)])
        return carry

    lax.fori_loop(0, ROWS_PER_TILE_SC // 16, zloop, 0)
    if with_deg:
        def zdloop(i, carry):
            pltpu.sync_copy(zblk.at[0], deg_sh.at[pl.ds(zbase + i * D, D)])
            return carry

        lax.fori_loop(0, ROWS_PER_TILE_SC // D, zdloop, 0)
    plsc.subcore_barrier()

    def fire(c, idxv, rowsv, sem):
        @pl.when(c < NCHUNK)
        def _():
            pltpu.sync_copy(edge_hbm.at[:, pl.ds(c * B, B)], idxv)
            pltpu.async_copy(x_hbm.at[idxv.at[0]], rowsv, sem)

    def drain(c, idxv, rowsv, sem):
        @pl.when(c < NCHUNK)
        def _():
            pltpu.make_async_copy(x_hbm.at[idxv.at[0]], rowsv, sem).wait()
            pltpu.sync_copy(rowsv, agg_sh.at[idxv.at[1]], add=True)
            if with_deg:
                pltpu.sync_copy(onesv, deg_sh.at[idxv.at[1]], add=True)

    fire(wid, idxA, rowsA, semA)

    def chunk_body(i, carry):
        c0 = wid + 2 * i * NW
        c1 = c0 + NW
        fire(c1, idxB, rowsB, semB)
        drain(c0, idxA, rowsA, semA)
        fire(c1 + NW, idxA, rowsA, semA)
        drain(c1, idxB, rowsB, semB)
        return carry

    lax.fori_loop(0, (CHUNKS_PER_TILE + 1) // 2, chunk_body, 0)
    plsc.subcore_barrier()

    pltpu.sync_copy(agg_sh.at[pl.ds(zbase, ROWS_PER_TILE_SC)],
                    agg_out.at[cid, pl.ds(zbase, ROWS_PER_TILE_SC)])
    if with_deg:
        pltpu.sync_copy(deg_sh.at[pl.ds(zbase, ROWS_PER_TILE_SC)],
                        deg_out.at[cid, pl.ds(zbase, ROWS_PER_TILE_SC)])


@functools.lru_cache(maxsize=None)
def _make_sc_agg(with_deg):
    mesh = plsc.VectorSubcoreMesh(core_axis_name="c", subcore_axis_name="s")
    out_type = [jax.ShapeDtypeStruct((NC, N_PAD, D), jnp.float32)]
    if with_deg:
        out_type.append(jax.ShapeDtypeStruct((NC, N_PAD), jnp.float32))
    scratch = [
        pltpu.VMEM((2, B), jnp.int32),
        pltpu.VMEM((2, B), jnp.int32),
        pltpu.VMEM((B, D), jnp.float32),
        pltpu.VMEM((B, D), jnp.float32),
        pltpu.VMEM((16, D), jnp.float32),
        pltpu.VMEM((B,), jnp.float32),
        pltpu.VMEM_SHARED((N_PAD, D), jnp.float32),
        pltpu.VMEM_SHARED((N_PAD,), jnp.float32),
        pltpu.SemaphoreType.DMA,
        pltpu.SemaphoreType.DMA,
    ]
    return pl.kernel(
        functools.partial(_sc_agg_body, with_deg),
        mesh=mesh,
        out_type=out_type if with_deg else out_type[0],
        scratch_types=scratch,
    )


R_BLK = 1024


def _dense_body(relu, agg_ref, deg_ref, x_ref, wl_ref, b_ref, wr_ref, o_ref):
    agg = agg_ref[0] + agg_ref[1]
    deg = deg_ref[0] + deg_ref[1]
    inv = 1.0 / jnp.maximum(deg, 1.0)
    mean = (agg.reshape(R_BLK // 128, 128, D) * inv[:, :, None]).reshape(R_BLK, D)
    out = jnp.dot(mean, wl_ref[...], preferred_element_type=jnp.float32)
    out = out + jnp.dot(x_ref[...], wr_ref[...], preferred_element_type=jnp.float32)
    out = out + b_ref[...]
    if relu:
        out = jnp.maximum(out, 0.0)
    o_ref[...] = out


def _dense(aggp, degp3, x_pad, wlT, b2, wrT, relu):
    grid = (N_PAD // R_BLK,)
    return pl.pallas_call(
        functools.partial(_dense_body, relu),
        grid=grid,
        in_specs=[
            pl.BlockSpec((NC, R_BLK, D), lambda i: (0, i, 0)),
            pl.BlockSpec((NC, R_BLK // 128, 128), lambda i: (0, i, 0)),
            pl.BlockSpec((R_BLK, D), lambda i: (i, 0)),
            pl.BlockSpec((D, D), lambda i: (0, 0)),
            pl.BlockSpec((1, D), lambda i: (0, 0)),
            pl.BlockSpec((D, D), lambda i: (0, 0)),
        ],
        out_specs=pl.BlockSpec((R_BLK, D), lambda i: (i, 0)),
        out_shape=jax.ShapeDtypeStruct((N_PAD, D), jnp.float32),
    )(aggp, degp3, x_pad, wlT, b2, wrT)


def kernel(x, edge_index, W_l1, b_l1, W_r1, W_l2, b_l2, W_r2):
    x_pad = jnp.pad(x, ((0, N_PAD - N), (0, 0)))

    aggp1, degp = _make_sc_agg(True)(x_pad, edge_index)
    degp3 = degp.reshape(NC, N_PAD // 128, 128)
    h = _dense(aggp1, degp3, x_pad, W_l1.T, b_l1.reshape(1, D), W_r1.T, True)

    aggp2 = _make_sc_agg(False)(h, edge_index)
    out = _dense(aggp2, degp3, h, W_l2.T, b_l2.reshape(1, D), W_r2.T, False)
    return out[:N]
